# baseline (speedup 1.0000x reference)
import functools

import jax
import jax.numpy as jnp
from jax import lax
from jax.experimental import pallas as pl
from jax.experimental.pallas import tpu as pltpu
from jax.experimental.pallas import tpu_sc as plsc

_CHUNK = 800
_N_BATCH_CHUNKS = 1


def _sc_info():
    info = plsc.get_sparse_core_info()
    return info.num_cores, info.num_subcores


@functools.lru_cache(maxsize=None)
def _make_transpose(vocab: int, d: int):
    nc, ns = _sc_info()
    nw = nc * ns
    assert d == 32
    cols_per_step = 512
    n_full = vocab // cols_per_step
    tail = vocab % cols_per_step
    per_w = n_full // nw
    n_extra = n_full % nw
    assert per_w % 2 == 1 and tail % 16 == 0

    mesh = plsc.VectorSubcoreMesh(core_axis_name="c", subcore_axis_name="s")

    @functools.partial(
        pl.kernel,
        mesh=mesh,
        out_type=jax.ShapeDtypeStruct((vocab * d // 128, 128), jnp.float32),
        scratch_types=[
            pltpu.VMEM((d, cols_per_step), jnp.float32),
            pltpu.VMEM((d, cols_per_step), jnp.float32),
            pltpu.VMEM((cols_per_step // 4, 128), jnp.float32),
            pltpu.VMEM((cols_per_step // 4, 128), jnp.float32),
            pltpu.SemaphoreType.DMA,
            pltpu.SemaphoreType.DMA,
            pltpu.SemaphoreType.DMA,
            pltpu.SemaphoreType.DMA,
        ],
        compiler_params=pltpu.CompilerParams(
            use_tc_tiling_on_sc=True, needs_layout_passes=False),
    )
    def transpose_kernel(tT_hbm, tail_hbm, t128_hbm, blk0, blk1, dst0, dst1,
                         si0, si1, so0, so1):
        wid = lax.axis_index("s") * nc + lax.axis_index("c")
        ii = lax.iota(jnp.int32, 16)
        ir = ii >> 2
        ic = (ii & 3) << 5

        def col0_of(j):
            return pl.multiple_of((wid + nw * j) * cols_per_step,
                                  cols_per_step)

        def load(j, blk, sem):
            pltpu.async_copy(
                tT_hbm.at[:, pl.ds(col0_of(j), cols_per_step)], blk, sem)

        def scatter(blk, dst):
            def col(c, carry):
                for m0 in range(0, cols_per_step, 16):
                    vals = blk[c, pl.ds(m0, 16)]
                    plsc.store_scatter(dst, [ir + (m0 // 4), ic + c], vals)
                return carry
            lax.fori_loop(0, d, col, 0)

        def store(j, dst, sem):
            row0 = pl.multiple_of(col0_of(j) // 4, cols_per_step // 4)
            pltpu.async_copy(dst, t128_hbm.at[pl.ds(row0, 128), :], sem)

        load(0, blk0, si0)
        load(1, blk1, si1)

        def body(t, carry):
            pltpu.make_async_copy(
                tT_hbm.at[:, pl.ds(0, cols_per_step)], blk0, si0).wait()

            @pl.when(t >= 1)
            def _():
                pltpu.make_async_copy(
                    dst0, t128_hbm.at[pl.ds(0, 128), :], so0).wait()
            scatter(blk0, dst0)
            store(2 * t, dst0, so0)
            load(2 * t + 2, blk0, si0)

            pltpu.make_async_copy(
                tT_hbm.at[:, pl.ds(0, cols_per_step)], blk1, si1).wait()

            @pl.when(t >= 1)
            def _():
                pltpu.make_async_copy(
                    dst1, t128_hbm.at[pl.ds(0, 128), :], so1).wait()
            scatter(blk1, dst1)
            store(2 * t + 1, dst1, so1)

            @pl.when(t < per_w // 2 - 1)
            def _():
                load(2 * t + 3, blk1, si1)
            return carry

        lax.fori_loop(0, per_w // 2, body, 0)

        pltpu.make_async_copy(
            tT_hbm.at[:, pl.ds(0, cols_per_step)], blk0, si0).wait()
        pltpu.make_async_copy(dst0, t128_hbm.at[pl.ds(0, 128), :], so0).wait()
        scatter(blk0, dst0)
        store(per_w - 1, dst0, so0)
        pltpu.make_async_copy(dst1, t128_hbm.at[pl.ds(0, 128), :], so1).wait()
        pltpu.make_async_copy(dst0, t128_hbm.at[pl.ds(0, 128), :], so0).wait()

        @pl.when(wid < n_extra)
        def _():
            c0 = pl.multiple_of((nw * per_w + wid) * cols_per_step,
                                cols_per_step)
            pltpu.sync_copy(tT_hbm.at[:, pl.ds(c0, cols_per_step)], blk0)
            scatter(blk0, dst0)
            r0 = pl.multiple_of(c0 // 4, cols_per_step // 4)
            pltpu.sync_copy(dst0, t128_hbm.at[pl.ds(r0, 128), :])

        @pl.when(wid == n_extra)
        def _():
            r0 = n_full * cols_per_step // 4
            pltpu.sync_copy(tail_hbm, dst0.at[pl.ds(0, tail // 4), :])
            pltpu.sync_copy(dst0.at[pl.ds(0, tail // 4), :],
                            t128_hbm.at[pl.ds(r0, tail // 4), :])

    return transpose_kernel


@functools.lru_cache(maxsize=None)
def _make_gather(n_idx: int, d: int):
    nc, ns = _sc_info()
    nw = nc * ns
    assert n_idx % nw == 0
    per_w = n_idx // nw
    assert per_w % (2 * _CHUNK) == 0
    half = per_w // _CHUNK // 2

    mesh = plsc.VectorSubcoreMesh(core_axis_name="c", subcore_axis_name="s")

    @functools.partial(
        pl.kernel,
        mesh=mesh,
        out_type=jax.ShapeDtypeStruct((n_idx, d), jnp.float32),
        scratch_types=[
            pltpu.VMEM((_CHUNK,), jnp.int32),
            pltpu.VMEM((_CHUNK,), jnp.int32),
            pltpu.VMEM((_CHUNK, d), jnp.float32),
            pltpu.VMEM((_CHUNK, d), jnp.float32),
            pltpu.SemaphoreType.DMA,
            pltpu.SemaphoreType.DMA,
            pltpu.SemaphoreType.DMA,
            pltpu.SemaphoreType.DMA,
            pltpu.SemaphoreType.DMA,
            pltpu.SemaphoreType.DMA,
        ],
        compiler_params=pltpu.CompilerParams(
            use_tc_tiling_on_sc=False, needs_layout_passes=False),
    )
    def gather_kernel(idx_hbm, table_hbm, out_hbm, idx0, idx1, rows0, rows1,
                      si0, si1, sg0, sg1, ss0, ss1):
        wid = lax.axis_index("s") * nc + lax.axis_index("c")
        base = wid * per_w

        def idx_load(g, buf, sem):
            pltpu.async_copy(idx_hbm.at[pl.ds(base + g * _CHUNK, _CHUNK)],
                             buf, sem)

        def store(g, buf, sem):
            pltpu.async_copy(buf, out_hbm.at[pl.ds(base + g * _CHUNK, _CHUNK)],
                             sem)

        idx_load(0, idx0, si0)
        idx_load(1, idx1, si1)
        pltpu.make_async_copy(
            idx_hbm.at[pl.ds(base, _CHUNK)], idx0, si0).wait()
        pltpu.async_copy(table_hbm.at[idx0], rows0, sg0)

        def body(t, carry):
            @pl.when(t >= 1)
            def _():
                pltpu.make_async_copy(
                    rows1, out_hbm.at[pl.ds(base, _CHUNK)], ss1).wait()
            pltpu.make_async_copy(
                idx_hbm.at[pl.ds(base, _CHUNK)], idx1, si1).wait()
            pltpu.async_copy(table_hbm.at[idx1], rows1, sg1)
            pltpu.make_async_copy(table_hbm.at[idx0], rows0, sg0).wait()
            store(2 * t, rows0, ss0)

            @pl.when(t < half - 1)
            def _():
                idx_load(2 * t + 2, idx0, si0)

            @pl.when(t < half - 1)
            def _():
                pltpu.make_async_copy(
                    rows0, out_hbm.at[pl.ds(base, _CHUNK)], ss0).wait()
                pltpu.make_async_copy(
                    idx_hbm.at[pl.ds(base, _CHUNK)], idx0, si0).wait()
                pltpu.async_copy(table_hbm.at[idx0], rows0, sg0)
            pltpu.make_async_copy(table_hbm.at[idx1], rows1, sg1).wait()
            store(2 * t + 1, rows1, ss1)

            @pl.when(t < half - 1)
            def _():
                idx_load(2 * t + 3, idx1, si1)

            return carry

        lax.fori_loop(0, half, body, 0)

        pltpu.make_async_copy(rows0, out_hbm.at[pl.ds(base, _CHUNK)], ss0).wait()
        pltpu.make_async_copy(rows1, out_hbm.at[pl.ds(base, _CHUNK)], ss1).wait()

    return gather_kernel


def kernel(x, table):
    b, l = x.shape
    vocab, d = table.shape
    n_main = vocab // 512 * 512
    tail_rows = table[n_main:].reshape(-1, 128)
    t128 = _make_transpose(vocab, d)(table.T, tail_rows)
    t_lin = t128.reshape(vocab, d)
    cb = b // _N_BATCH_CHUNKS
    gather = _make_gather(cb * l, d)
    outs = []
    for i in range(_N_BATCH_CHUNKS):
        xi = x[i * cb:(i + 1) * cb].reshape(-1).astype(jnp.int32)
        oi = gather(xi, t_lin)
        outs.append(oi.reshape(cb, l * d))
    return jnp.concatenate(outs, axis=0)
